# baseline (speedup 1.0000x reference)
import functools

import jax
import jax.numpy as jnp
from jax import lax
from jax.experimental import pallas as pl
from jax.experimental.pallas import tpu as pltpu
from jax.experimental.pallas import tpu_sc as plsc

N = 10000
E = 160000
D = 256
NTILES = 32
NPT = 320
N_PAD = NTILES * NPT
CH = 2000
FLUSH = 4096
BUF = FLUSH + 16
E_PAD = 40 * FLUSH
K = 32
NBUF = 4
SPAN = 2048
BM = 1000

def _partition_body(src_in, dst_in, ew, counts, esrc, edstl, eww,
                    srcc, dstc, wc, bsrc, bdst, bw, cbuf):
    wid = lax.axis_index("s") * 2 + lax.axis_index("c")
    base = wid * NPT
    ebase = wid * E_PAD

    zi = jnp.zeros((16,), jnp.int32)
    zf = jnp.zeros((16,), jnp.float32)

    def zero_b(i, _):
        bsrc[pl.ds(i * 16, 16)] = zi
        bdst[pl.ds(i * 16, 16)] = zi
        bw[pl.ds(i * 16, 16)] = zf
        return 0

    lax.fori_loop(0, BUF // 16, zero_b, 0)

    def chunk(c, carry):
        pltpu.sync_copy(src_in.at[pl.ds(c * CH, CH)], srcc)
        pltpu.sync_copy(dst_in.at[pl.ds(c * CH, CH)], dstc)
        pltpu.sync_copy(ew.at[pl.ds(c * CH, CH)], wc)

        def vstep(j, cy):
            cnt, hb = cy
            vd = dstc[pl.ds(j * 16, 16)]
            off = vd - base
            m = (off >= 0) & (off < NPT)
            nmatch = plsc.all_reduce_population_count(m)[0]
            vs = srcc[pl.ds(j * 16, 16)]
            vw = wc[pl.ds(j * 16, 16)]
            plsc.store_compressed(bsrc.at[pl.ds(cnt, 16)], vs, mask=m)
            plsc.store_compressed(bdst.at[pl.ds(cnt, 16)], off, mask=m)
            plsc.store_compressed(bw.at[pl.ds(cnt, 16)], vw, mask=m)
            cnt = cnt + nmatch

            def flush(cy3):
                    cnt2, hb2 = cy3
                    pltpu.sync_copy(bsrc.at[pl.ds(0, FLUSH)],
                                    esrc.at[pl.ds(pl.multiple_of(ebase + hb2, FLUSH), FLUSH)])
                    pltpu.sync_copy(bdst.at[pl.ds(0, FLUSH)],
                                    edstl.at[pl.ds(pl.multiple_of(ebase + hb2, FLUSH), FLUSH)])
                    pltpu.sync_copy(bw.at[pl.ds(0, FLUSH)],
                                    eww.at[pl.ds(pl.multiple_of(ebase + hb2, FLUSH), FLUSH)])
                    bsrc[pl.ds(0, 16)] = bsrc[pl.ds(FLUSH, 16)]
                    bdst[pl.ds(0, 16)] = bdst[pl.ds(FLUSH, 16)]
                    bw[pl.ds(0, 16)] = bw[pl.ds(FLUSH, 16)]
                    return (cnt2 - FLUSH, hb2 + FLUSH)

            return lax.cond(cnt >= FLUSH, flush, lambda z: z, (cnt, hb))

        return lax.fori_loop(0, CH // 16, vstep, carry)

    cnt, hb = lax.fori_loop(0, E // CH, chunk,
                            (jnp.int32(0), jnp.int32(0)))

    pltpu.sync_copy(bsrc.at[pl.ds(0, FLUSH)], esrc.at[pl.ds(pl.multiple_of(ebase + hb, FLUSH), FLUSH)])
    pltpu.sync_copy(bdst.at[pl.ds(0, FLUSH)], edstl.at[pl.ds(pl.multiple_of(ebase + hb, FLUSH), FLUSH)])
    pltpu.sync_copy(bw.at[pl.ds(0, FLUSH)], eww.at[pl.ds(pl.multiple_of(ebase + hb, FLUSH), FLUSH)])
    cbuf[...] = jnp.full((16,), hb + cnt, jnp.int32)
    pltpu.sync_copy(cbuf, counts.at[pl.ds(pl.multiple_of(wid * 16, 16), 16)])


def _segmax_body(p, counts, esrc, edstl, eww, pooled,
                 acc, bufs, sidx, sdst, sw, cbuf, sems):
    wid = lax.axis_index("s") * 2 + lax.axis_index("c")
    base = wid * NPT
    ebase = wid * E_PAD
    pltpu.sync_copy(counts.at[pl.ds(pl.multiple_of(wid * 16, 16), 16)], cbuf)
    count = cbuf[...][0]

    zf = jnp.zeros((16,), jnp.float32)

    def zr(r, _):
        for ci in range(D // 16):
            acc[r, pl.ds(ci * 16, 16)] = zf
        return 0

    lax.fori_loop(0, NPT, zr, 0)

    def accum(rows, ebeg):
        def grp(g, _):
            dv = sdst[pl.ds(ebeg + g * 16, 16)]
            w16 = sw[pl.ds(ebeg + g * 16, 16)]
            for lane in range(16):
                d = dv[lane]
                w = w16[lane]
                for ci in range(D // 16):
                    sl = pl.ds(ci * 16, 16)
                    acc[d, sl] = jnp.maximum(
                        acc[d, sl], rows[g * 16 + lane, sl] * w)
            return 0

        lax.fori_loop(0, K // 16, grp, 0)

    def issue(t, b):
        off = pl.multiple_of(jnp.minimum(t, SPAN // K - 1) * K, K)
        pltpu.async_copy(p.at[sidx.at[pl.ds(off, K)]], bufs.at[b], sems.at[b])

    def drain(b):
        pltpu.make_async_copy(p.at[pl.ds(0, K)], bufs.at[b], sems.at[b]).wait()

    nspan = (count + (SPAN - 1)) // SPAN

    def span(s, _):
        soff = pl.multiple_of(ebase + s * SPAN, SPAN)
        pltpu.sync_copy(esrc.at[pl.ds(soff, SPAN)], sidx)
        pltpu.sync_copy(edstl.at[pl.ds(soff, SPAN)], sdst)
        pltpu.sync_copy(eww.at[pl.ds(soff, SPAN)], sw)
        rem = jnp.minimum(count - s * SPAN, SPAN)
        ng = (rem + (NBUF * K - 1)) // (NBUF * K)
        for b in range(NBUF):
            issue(jnp.int32(b), b)

        def ring(g, _):
            for b in range(NBUF):
                t = g * NBUF + b
                drain(b)
                accum(bufs.at[b], t * K)
                issue(t + NBUF, b)
            return 0

        lax.fori_loop(0, ng, ring, 0)
        for b in range(NBUF):
            drain(b)
        return 0

    lax.fori_loop(0, nspan, span, 0)
    pltpu.sync_copy(acc, pooled.at[pl.ds(base, NPT)])


@functools.lru_cache(maxsize=1)
def _build_sc():
    mesh = plsc.VectorSubcoreMesh(core_axis_name="c", subcore_axis_name="s",
                                  num_cores=2, num_subcores=16)
    sc_params = pltpu.CompilerParams(needs_layout_passes=False)
    partition = pl.kernel(
        _partition_body,
        out_type=(
            jax.ShapeDtypeStruct((NTILES * 16,), jnp.int32),
            jax.ShapeDtypeStruct((NTILES * E_PAD,), jnp.int32),
            jax.ShapeDtypeStruct((NTILES * E_PAD,), jnp.int32),
            jax.ShapeDtypeStruct((NTILES * E_PAD,), jnp.float32),
        ),
        mesh=mesh,
        scratch_types=[
            pltpu.VMEM((CH,), jnp.int32),
            pltpu.VMEM((CH,), jnp.int32),
            pltpu.VMEM((CH,), jnp.float32),
            pltpu.VMEM((BUF,), jnp.int32),
            pltpu.VMEM((BUF,), jnp.int32),
            pltpu.VMEM((BUF,), jnp.float32),
            pltpu.VMEM((16,), jnp.int32),
        ],
        compiler_params=sc_params,
    )
    segmax = pl.kernel(
        _segmax_body,
        out_type=jax.ShapeDtypeStruct((N_PAD, D), jnp.float32),
        mesh=mesh,
        scratch_types=[
            pltpu.VMEM((NPT, D), jnp.float32),
            pltpu.VMEM((NBUF, K, D), jnp.float32),
            pltpu.VMEM((SPAN,), jnp.int32),
            pltpu.VMEM((SPAN,), jnp.int32),
            pltpu.VMEM((SPAN,), jnp.float32),
            pltpu.VMEM((16,), jnp.int32),
            pltpu.SemaphoreType.DMA((NBUF,)),
        ],
        compiler_params=sc_params,
    )
    return partition, segmax


def _dotT(a, w):
    return lax.dot_general(a, w, (((1,), (1,)), ((), ())),
                           preferred_element_type=jnp.float32)


def _tc1_body(x_ref, w1_ref, b1_ref, wp_ref, bp_ref, ws_ref, bl_ref,
              h1_ref, p0_ref, s0_ref):
    h1 = jnp.tanh(_dotT(x_ref[...], w1_ref[...]) + b1_ref[0:1, :])
    p0 = jnp.maximum(_dotT(h1, wp_ref[...]) + bp_ref[0:1, :], 0.0)
    s0 = _dotT(h1, ws_ref[...]) + bl_ref[0:1, :]
    h1_ref[...] = h1
    p0_ref[...] = p0
    s0_ref[...] = s0


def _tc2_body(h1_ref, s0_ref, pooled_ref, wn_ref, wp_ref, bp_ref,
              ws_ref, bl_ref, h2_ref, p1_ref, s1_ref):
    h2 = h1_ref[...] + jnp.tanh(s0_ref[...] + _dotT(pooled_ref[...], wn_ref[...]))
    p1 = jnp.maximum(_dotT(h2, wp_ref[...]) + bp_ref[0:1, :], 0.0)
    s1 = _dotT(h2, ws_ref[...]) + bl_ref[0:1, :]
    h2_ref[...] = h2
    p1_ref[...] = p1
    s1_ref[...] = s1


def _tc3_body(h2_ref, s1_ref, pooled_ref, wn_ref, w2_ref, b2_ref,
              out_ref, mask_ref):
    h3 = h2_ref[...] + s1_ref[...] + _dotT(pooled_ref[...], wn_ref[...])
    out8 = _dotT(jnp.tanh(h3), w2_ref[...]) + b2_ref[0:1, :]
    allz = jnp.all(h3 == 0.0, axis=1, keepdims=True)
    out_ref[...] = out8
    mask_ref[...] = jnp.broadcast_to(allz, (BM, 8)).astype(jnp.int32)


def _row_spec():
    return pl.BlockSpec((BM, D), lambda m: (m, 0))


def _full_spec(shape):
    return pl.BlockSpec(shape, lambda m: tuple(0 for _ in shape))


_tc1 = pl.pallas_call(
    _tc1_body,
    grid=(N // BM,),
    in_specs=[_row_spec(), _full_spec((D, D)), _full_spec((8, D)),
              _full_spec((D, D)), _full_spec((8, D)),
              _full_spec((D, D)), _full_spec((8, D))],
    out_specs=[_row_spec(), _row_spec(), _row_spec()],
    out_shape=[jax.ShapeDtypeStruct((N, D), jnp.float32)] * 3,
)

_tc2 = pl.pallas_call(
    _tc2_body,
    grid=(N // BM,),
    in_specs=[_row_spec(), _row_spec(), _row_spec(),
              _full_spec((D, D)), _full_spec((D, D)), _full_spec((8, D)),
              _full_spec((D, D)), _full_spec((8, D))],
    out_specs=[_row_spec(), _row_spec(), _row_spec()],
    out_shape=[jax.ShapeDtypeStruct((N, D), jnp.float32)] * 3,
)

_tc3 = pl.pallas_call(
    _tc3_body,
    grid=(N // BM,),
    in_specs=[_row_spec(), _row_spec(), _row_spec(),
              _full_spec((D, D)), _full_spec((8, D)), _full_spec((8, 8))],
    out_specs=[pl.BlockSpec((BM, 8), lambda m: (m, 0)),
               pl.BlockSpec((BM, 8), lambda m: (m, 0))],
    out_shape=[jax.ShapeDtypeStruct((N, 8), jnp.float32),
               jax.ShapeDtypeStruct((N, 8), jnp.int32)],
)


def _pad_rows(v, rows=8):
    return jnp.broadcast_to(v.reshape(1, -1), (rows, v.shape[0]))


def kernel(x, edge_index, edge_weight, W1, b1, Wp0, bp0, Ws0, Wn0, bl0,
           Wp1, bp1, Ws1, Wn1, bl1, W2, b2):
    _partition, _segmax = _build_sc()
    counts, esrc, edstl, eww = _partition(edge_index[0], edge_index[1],
                                          edge_weight)

    b1p, bp0p, bl0p = _pad_rows(b1), _pad_rows(bp0), _pad_rows(bl0)
    bp1p, bl1p = _pad_rows(bp1), _pad_rows(bl1)
    w2p = jnp.broadcast_to(W2, (8, D))
    b2p = jnp.broadcast_to(b2.reshape(1, 1), (8, 8))

    h1, p0, s0 = _tc1(x, W1, b1p, Wp0, bp0p, Ws0, bl0p)
    pooled0 = _segmax(p0, counts, esrc, edstl, eww)
    h2, p1, s1 = _tc2(h1, s0, pooled0, Wn0, Wp1, bp1p, Ws1, bl1p)
    pooled1 = _segmax(p1, counts, esrc, edstl, eww)
    out8, mask8 = _tc3(h2, s1, pooled1, Wn1, w2p, b2p)

    return out8[:, 0:1], mask8[:, 0].astype(bool)


# speedup vs baseline: 1.3863x; 1.3863x over previous
import functools

import jax
import jax.numpy as jnp
from jax import lax
from jax.experimental import pallas as pl
from jax.experimental.pallas import tpu as pltpu
from jax.experimental.pallas import tpu_sc as plsc

N = 10000
E = 160000
D = 256
NTILES = 32
NPT = 320
N_PAD = NTILES * NPT
CH = 2000
FLUSH = 4096
BUF = FLUSH + 16
E_PAD = 40 * FLUSH
K = 64
NBUF = 2
SPAN = 2048
BM = 1000

def _partition_body(src_in, dst_in, ew, counts, esrc, edstl, eww,
                    srcc, dstc, wc, bsrc, bdst, bw, cbuf):
    wid = lax.axis_index("s") * 2 + lax.axis_index("c")
    base = wid * NPT
    ebase = wid * E_PAD

    zi = jnp.zeros((16,), jnp.int32)
    zf = jnp.zeros((16,), jnp.float32)

    def zero_b(i, _):
        bsrc[pl.ds(i * 16, 16)] = zi
        bdst[pl.ds(i * 16, 16)] = zi
        bw[pl.ds(i * 16, 16)] = zf
        return 0

    lax.fori_loop(0, BUF // 16, zero_b, 0)

    def chunk(c, carry):
        pltpu.sync_copy(src_in.at[pl.ds(c * CH, CH)], srcc)
        pltpu.sync_copy(dst_in.at[pl.ds(c * CH, CH)], dstc)
        pltpu.sync_copy(ew.at[pl.ds(c * CH, CH)], wc)

        def vstep(j, cy):
            cnt, hb = cy
            vd = dstc[pl.ds(j * 16, 16)]
            off = vd - base
            m = (off >= 0) & (off < NPT)
            nmatch = plsc.all_reduce_population_count(m)[0]
            vs = srcc[pl.ds(j * 16, 16)]
            vw = wc[pl.ds(j * 16, 16)]
            plsc.store_compressed(bsrc.at[pl.ds(cnt, 16)], vs, mask=m)
            plsc.store_compressed(bdst.at[pl.ds(cnt, 16)], off, mask=m)
            plsc.store_compressed(bw.at[pl.ds(cnt, 16)], vw, mask=m)
            cnt = cnt + nmatch

            def flush(cy3):
                    cnt2, hb2 = cy3
                    pltpu.sync_copy(bsrc.at[pl.ds(0, FLUSH)],
                                    esrc.at[pl.ds(pl.multiple_of(ebase + hb2, FLUSH), FLUSH)])
                    pltpu.sync_copy(bdst.at[pl.ds(0, FLUSH)],
                                    edstl.at[pl.ds(pl.multiple_of(ebase + hb2, FLUSH), FLUSH)])
                    pltpu.sync_copy(bw.at[pl.ds(0, FLUSH)],
                                    eww.at[pl.ds(pl.multiple_of(ebase + hb2, FLUSH), FLUSH)])
                    bsrc[pl.ds(0, 16)] = bsrc[pl.ds(FLUSH, 16)]
                    bdst[pl.ds(0, 16)] = bdst[pl.ds(FLUSH, 16)]
                    bw[pl.ds(0, 16)] = bw[pl.ds(FLUSH, 16)]
                    return (cnt2 - FLUSH, hb2 + FLUSH)

            return lax.cond(cnt >= FLUSH, flush, lambda z: z, (cnt, hb))

        return lax.fori_loop(0, CH // 16, vstep, carry)

    cnt, hb = lax.fori_loop(0, E // CH, chunk,
                            (jnp.int32(0), jnp.int32(0)))

    pltpu.sync_copy(bsrc.at[pl.ds(0, FLUSH)], esrc.at[pl.ds(pl.multiple_of(ebase + hb, FLUSH), FLUSH)])
    pltpu.sync_copy(bdst.at[pl.ds(0, FLUSH)], edstl.at[pl.ds(pl.multiple_of(ebase + hb, FLUSH), FLUSH)])
    pltpu.sync_copy(bw.at[pl.ds(0, FLUSH)], eww.at[pl.ds(pl.multiple_of(ebase + hb, FLUSH), FLUSH)])
    cbuf[...] = jnp.full((16,), hb + cnt, jnp.int32)
    pltpu.sync_copy(cbuf, counts.at[pl.ds(pl.multiple_of(wid * 16, 16), 16)])


def _segmax_body(p, counts, esrc, edstl, eww, pooled,
                 acc, bufs, sidx, sdst, sw, cbuf, sems):
    wid = lax.axis_index("s") * 2 + lax.axis_index("c")
    base = wid * NPT
    ebase = wid * E_PAD
    pltpu.sync_copy(counts.at[pl.ds(pl.multiple_of(wid * 16, 16), 16)], cbuf)
    count = cbuf[...][0]

    zf = jnp.zeros((16,), jnp.float32)

    def zr(r, _):
        for ci in range(D // 16):
            acc[r, pl.ds(ci * 16, 16)] = zf
        return 0

    lax.fori_loop(0, NPT, zr, 0)

    def accum(rows, ebeg):
        def grp(g, _):
            dv = sdst[pl.ds(ebeg + g * 16, 16)]
            w16 = sw[pl.ds(ebeg + g * 16, 16)]
            for lane in range(16):
                d = dv[lane]
                w = w16[lane]
                for ci in range(D // 16):
                    sl = pl.ds(ci * 16, 16)
                    acc[d, sl] = jnp.maximum(
                        acc[d, sl], rows[g * 16 + lane, sl] * w)
            return 0

        lax.fori_loop(0, K // 16, grp, 0)

    def issue(t, b):
        off = pl.multiple_of(jnp.minimum(t, SPAN // K - 1) * K, K)
        pltpu.async_copy(p.at[sidx.at[pl.ds(off, K)]], bufs.at[b], sems.at[b])

    def drain(b):
        pltpu.make_async_copy(p.at[pl.ds(0, K)], bufs.at[b], sems.at[b]).wait()

    nspan = (count + (SPAN - 1)) // SPAN

    def span(s, _):
        soff = pl.multiple_of(ebase + s * SPAN, SPAN)
        pltpu.sync_copy(esrc.at[pl.ds(soff, SPAN)], sidx)
        pltpu.sync_copy(edstl.at[pl.ds(soff, SPAN)], sdst)
        pltpu.sync_copy(eww.at[pl.ds(soff, SPAN)], sw)
        rem = jnp.minimum(count - s * SPAN, SPAN)
        ng = (rem + (NBUF * K - 1)) // (NBUF * K)
        for b in range(NBUF):
            issue(jnp.int32(b), b)

        def ring(g, _):
            for b in range(NBUF):
                t = g * NBUF + b
                drain(b)
                accum(bufs.at[b], t * K)
                issue(t + NBUF, b)
            return 0

        lax.fori_loop(0, ng, ring, 0)
        for b in range(NBUF):
            drain(b)
        return 0

    lax.fori_loop(0, nspan, span, 0)
    pltpu.sync_copy(acc, pooled.at[pl.ds(base, NPT)])


@functools.lru_cache(maxsize=1)
def _build_sc():
    mesh = plsc.VectorSubcoreMesh(core_axis_name="c", subcore_axis_name="s",
                                  num_cores=2, num_subcores=16)
    sc_params = pltpu.CompilerParams(needs_layout_passes=False)
    partition = pl.kernel(
        _partition_body,
        out_type=(
            jax.ShapeDtypeStruct((NTILES * 16,), jnp.int32),
            jax.ShapeDtypeStruct((NTILES * E_PAD,), jnp.int32),
            jax.ShapeDtypeStruct((NTILES * E_PAD,), jnp.int32),
            jax.ShapeDtypeStruct((NTILES * E_PAD,), jnp.float32),
        ),
        mesh=mesh,
        scratch_types=[
            pltpu.VMEM((CH,), jnp.int32),
            pltpu.VMEM((CH,), jnp.int32),
            pltpu.VMEM((CH,), jnp.float32),
            pltpu.VMEM((BUF,), jnp.int32),
            pltpu.VMEM((BUF,), jnp.int32),
            pltpu.VMEM((BUF,), jnp.float32),
            pltpu.VMEM((16,), jnp.int32),
        ],
        compiler_params=sc_params,
    )
    segmax = pl.kernel(
        _segmax_body,
        out_type=jax.ShapeDtypeStruct((N_PAD, D), jnp.float32),
        mesh=mesh,
        scratch_types=[
            pltpu.VMEM((NPT, D), jnp.float32),
            pltpu.VMEM((NBUF, K, D), jnp.float32),
            pltpu.VMEM((SPAN,), jnp.int32),
            pltpu.VMEM((SPAN,), jnp.int32),
            pltpu.VMEM((SPAN,), jnp.float32),
            pltpu.VMEM((16,), jnp.int32),
            pltpu.SemaphoreType.DMA((NBUF,)),
        ],
        compiler_params=sc_params,
    )
    return partition, segmax


def _dotT(a, w):
    return lax.dot_general(a, w, (((1,), (1,)), ((), ())),
                           preferred_element_type=jnp.float32)


def _tc1_body(x_ref, w1_ref, b1_ref, wp_ref, bp_ref, ws_ref, bl_ref,
              h1_ref, p0_ref, s0_ref):
    h1 = jnp.tanh(_dotT(x_ref[...], w1_ref[...]) + b1_ref[0:1, :])
    p0 = jnp.maximum(_dotT(h1, wp_ref[...]) + bp_ref[0:1, :], 0.0)
    s0 = _dotT(h1, ws_ref[...]) + bl_ref[0:1, :]
    h1_ref[...] = h1
    p0_ref[...] = p0
    s0_ref[...] = s0


def _tc2_body(h1_ref, s0_ref, pooled_ref, wn_ref, wp_ref, bp_ref,
              ws_ref, bl_ref, h2_ref, p1_ref, s1_ref):
    h2 = h1_ref[...] + jnp.tanh(s0_ref[...] + _dotT(pooled_ref[...], wn_ref[...]))
    p1 = jnp.maximum(_dotT(h2, wp_ref[...]) + bp_ref[0:1, :], 0.0)
    s1 = _dotT(h2, ws_ref[...]) + bl_ref[0:1, :]
    h2_ref[...] = h2
    p1_ref[...] = p1
    s1_ref[...] = s1


def _tc3_body(h2_ref, s1_ref, pooled_ref, wn_ref, w2_ref, b2_ref,
              out_ref, mask_ref):
    h3 = h2_ref[...] + s1_ref[...] + _dotT(pooled_ref[...], wn_ref[...])
    out8 = _dotT(jnp.tanh(h3), w2_ref[...]) + b2_ref[0:1, :]
    allz = jnp.all(h3 == 0.0, axis=1, keepdims=True)
    out_ref[...] = out8
    mask_ref[...] = jnp.broadcast_to(allz, (BM, 8)).astype(jnp.int32)


def _row_spec():
    return pl.BlockSpec((BM, D), lambda m: (m, 0))


def _full_spec(shape):
    return pl.BlockSpec(shape, lambda m: tuple(0 for _ in shape))


_tc1 = pl.pallas_call(
    _tc1_body,
    grid=(N // BM,),
    in_specs=[_row_spec(), _full_spec((D, D)), _full_spec((8, D)),
              _full_spec((D, D)), _full_spec((8, D)),
              _full_spec((D, D)), _full_spec((8, D))],
    out_specs=[_row_spec(), _row_spec(), _row_spec()],
    out_shape=[jax.ShapeDtypeStruct((N, D), jnp.float32)] * 3,
)

_tc2 = pl.pallas_call(
    _tc2_body,
    grid=(N // BM,),
    in_specs=[_row_spec(), _row_spec(), _row_spec(),
              _full_spec((D, D)), _full_spec((D, D)), _full_spec((8, D)),
              _full_spec((D, D)), _full_spec((8, D))],
    out_specs=[_row_spec(), _row_spec(), _row_spec()],
    out_shape=[jax.ShapeDtypeStruct((N, D), jnp.float32)] * 3,
)

_tc3 = pl.pallas_call(
    _tc3_body,
    grid=(N // BM,),
    in_specs=[_row_spec(), _row_spec(), _row_spec(),
              _full_spec((D, D)), _full_spec((8, D)), _full_spec((8, 8))],
    out_specs=[pl.BlockSpec((BM, 8), lambda m: (m, 0)),
               pl.BlockSpec((BM, 8), lambda m: (m, 0))],
    out_shape=[jax.ShapeDtypeStruct((N, 8), jnp.float32),
               jax.ShapeDtypeStruct((N, 8), jnp.int32)],
)


def _pad_rows(v, rows=8):
    return jnp.broadcast_to(v.reshape(1, -1), (rows, v.shape[0]))


def kernel(x, edge_index, edge_weight, W1, b1, Wp0, bp0, Ws0, Wn0, bl0,
           Wp1, bp1, Ws1, Wn1, bl1, W2, b2):
    _partition, _segmax = _build_sc()
    counts, esrc, edstl, eww = _partition(edge_index[0], edge_index[1],
                                          edge_weight)

    b1p, bp0p, bl0p = _pad_rows(b1), _pad_rows(bp0), _pad_rows(bl0)
    bp1p, bl1p = _pad_rows(bp1), _pad_rows(bl1)
    w2p = jnp.broadcast_to(W2, (8, D))
    b2p = jnp.broadcast_to(b2.reshape(1, 1), (8, 8))

    h1, p0, s0 = _tc1(x, W1, b1p, Wp0, bp0p, Ws0, bl0p)
    pooled0 = _segmax(p0, counts, esrc, edstl, eww)
    h2, p1, s1 = _tc2(h1, s0, pooled0, Wn0, Wp1, bp1p, Ws1, bl1p)
    pooled1 = _segmax(p1, counts, esrc, edstl, eww)
    out8, mask8 = _tc3(h2, s1, pooled1, Wn1, w2p, b2p)

    return out8[:, 0:1], mask8[:, 0].astype(bool)


# speedup vs baseline: 2.0899x; 1.5075x over previous
import functools

import jax
import jax.numpy as jnp
from jax import lax
from jax.experimental import pallas as pl
from jax.experimental.pallas import tpu as pltpu
from jax.experimental.pallas import tpu_sc as plsc

N = 10000
E = 160000
D = 256
NTILES = 32
NPT = 320
N_PAD = NTILES * NPT
CH = 2000
FLUSH = 4096
BUF = FLUSH + 16
E_PAD = 40 * FLUSH
K = 128
DP = 128
NBUF = 2
SPAN = 2048
BM = 1000

def _partition_body(src_in, dst_in, ew, counts, esrc, edstl, eww,
                    srcc, dstc, wc, bsrc, bdst, bw, cbuf):
    wid = lax.axis_index("s") * 2 + lax.axis_index("c")
    base = wid * NPT
    ebase = wid * E_PAD

    zi = jnp.zeros((16,), jnp.int32)
    zf = jnp.zeros((16,), jnp.float32)

    def zero_b(i, _):
        bsrc[pl.ds(i * 16, 16)] = zi
        bdst[pl.ds(i * 16, 16)] = zi
        bw[pl.ds(i * 16, 16)] = zf
        return 0

    lax.fori_loop(0, BUF // 16, zero_b, 0)

    def chunk(c, carry):
        pltpu.sync_copy(src_in.at[pl.ds(c * CH, CH)], srcc)
        pltpu.sync_copy(dst_in.at[pl.ds(c * CH, CH)], dstc)
        pltpu.sync_copy(ew.at[pl.ds(c * CH, CH)], wc)

        def vstep(j, cy):
            cnt, hb = cy
            vd = dstc[pl.ds(j * 16, 16)]
            off = vd - base
            m = (off >= 0) & (off < NPT)
            nmatch = plsc.all_reduce_population_count(m)[0]
            vs = srcc[pl.ds(j * 16, 16)]
            vw = wc[pl.ds(j * 16, 16)]
            plsc.store_compressed(bsrc.at[pl.ds(cnt, 16)], vs, mask=m)
            plsc.store_compressed(bdst.at[pl.ds(cnt, 16)], off, mask=m)
            plsc.store_compressed(bw.at[pl.ds(cnt, 16)], vw, mask=m)
            cnt = cnt + nmatch

            def flush(cy3):
                    cnt2, hb2 = cy3
                    pltpu.sync_copy(bsrc.at[pl.ds(0, FLUSH)],
                                    esrc.at[pl.ds(pl.multiple_of(ebase + hb2, FLUSH), FLUSH)])
                    pltpu.sync_copy(bdst.at[pl.ds(0, FLUSH)],
                                    edstl.at[pl.ds(pl.multiple_of(ebase + hb2, FLUSH), FLUSH)])
                    pltpu.sync_copy(bw.at[pl.ds(0, FLUSH)],
                                    eww.at[pl.ds(pl.multiple_of(ebase + hb2, FLUSH), FLUSH)])
                    bsrc[pl.ds(0, 16)] = bsrc[pl.ds(FLUSH, 16)]
                    bdst[pl.ds(0, 16)] = bdst[pl.ds(FLUSH, 16)]
                    bw[pl.ds(0, 16)] = bw[pl.ds(FLUSH, 16)]
                    return (cnt2 - FLUSH, hb2 + FLUSH)

            return lax.cond(cnt >= FLUSH, flush, lambda z: z, (cnt, hb))

        return lax.fori_loop(0, CH // 16, vstep, carry)

    cnt, hb = lax.fori_loop(0, E // CH, chunk,
                            (jnp.int32(0), jnp.int32(0)))

    pltpu.sync_copy(bsrc.at[pl.ds(0, FLUSH)], esrc.at[pl.ds(pl.multiple_of(ebase + hb, FLUSH), FLUSH)])
    pltpu.sync_copy(bdst.at[pl.ds(0, FLUSH)], edstl.at[pl.ds(pl.multiple_of(ebase + hb, FLUSH), FLUSH)])
    pltpu.sync_copy(bw.at[pl.ds(0, FLUSH)], eww.at[pl.ds(pl.multiple_of(ebase + hb, FLUSH), FLUSH)])
    cbuf[...] = jnp.full((16,), hb + cnt, jnp.int32)
    pltpu.sync_copy(cbuf, counts.at[pl.ds(pl.multiple_of(wid * 16, 16), 16)])


def _segmax_body(p, counts, esrc, edstl, eww, pooled,
                 acc, bufs, sidx, sdst, sw, cbuf, sems):
    wid = lax.axis_index("s") * 2 + lax.axis_index("c")
    base = wid * NPT
    ebase = wid * E_PAD
    pltpu.sync_copy(counts.at[pl.ds(pl.multiple_of(wid * 16, 16), 16)], cbuf)
    count = cbuf[...][0]

    zf = jnp.zeros((16,), jnp.float32)

    def zr(r, _):
        for ci in range(D // 16):
            acc[r, pl.ds(ci * 16, 16)] = zf
        return 0

    lax.fori_loop(0, NPT, zr, 0)

    def accum(rows, ebeg):
        def grp(g, _):
            dv = sdst[pl.ds(ebeg + g * 16, 16)]
            w16 = sw[pl.ds(ebeg + g * 16, 16)]
            for lane in range(16):
                d = dv[lane]
                w = w16[lane]
                for ci in range(DP // 16):
                    v = rows[g * 16 + lane, pl.ds(ci * 16, 16)]
                    ra = plsc.bitcast(v << 16, jnp.float32)
                    rb = plsc.bitcast(v & jnp.int32(-65536), jnp.float32)
                    sa = pl.ds(ci * 16, 16)
                    sb = pl.ds(DP + ci * 16, 16)
                    acc[d, sa] = jnp.maximum(acc[d, sa], ra * w)
                    acc[d, sb] = jnp.maximum(acc[d, sb], rb * w)
            return 0

        lax.fori_loop(0, K // 16, grp, 0)

    def issue(t, b):
        off = pl.multiple_of(jnp.minimum(t, SPAN // K - 1) * K, K)
        pltpu.async_copy(p.at[sidx.at[pl.ds(off, K)]], bufs.at[b], sems.at[b])

    def drain(b):
        pltpu.make_async_copy(p.at[pl.ds(0, K)], bufs.at[b], sems.at[b]).wait()

    nspan = (count + (SPAN - 1)) // SPAN

    def span(s, _):
        soff = pl.multiple_of(ebase + s * SPAN, SPAN)
        pltpu.sync_copy(esrc.at[pl.ds(soff, SPAN)], sidx)
        pltpu.sync_copy(edstl.at[pl.ds(soff, SPAN)], sdst)
        pltpu.sync_copy(eww.at[pl.ds(soff, SPAN)], sw)
        rem = jnp.minimum(count - s * SPAN, SPAN)
        ng = (rem + (NBUF * K - 1)) // (NBUF * K)
        for b in range(NBUF):
            issue(jnp.int32(b), b)

        def ring(g, _):
            for b in range(NBUF):
                t = g * NBUF + b
                drain(b)
                accum(bufs.at[b], t * K)
                issue(t + NBUF, b)
            return 0

        lax.fori_loop(0, ng, ring, 0)
        for b in range(NBUF):
            drain(b)
        return 0

    lax.fori_loop(0, nspan, span, 0)

    WR = 64

    def rnd16(x):
        u = plsc.bitcast(x, jnp.int32)
        return (u + jnp.int32(0x7FFF)
                + (lax.shift_right_logical(u, 16) & 1)) >> 16

    for wchunk in range(NPT // WR):

        def wrow(r2, _):
            for ci in range(DP // 16):
                lo = rnd16(acc[wchunk * WR + r2, pl.ds(ci * 16, 16)])
                hi = rnd16(acc[wchunk * WR + r2, pl.ds(DP + ci * 16, 16)])
                bufs[0, r2, pl.ds(ci * 16, 16)] = lo | (hi << 16)
            return 0

        lax.fori_loop(0, WR, wrow, 0)
        pltpu.sync_copy(bufs.at[0].at[pl.ds(0, WR)],
                        pooled.at[pl.ds(base + wchunk * WR, WR)])


@functools.lru_cache(maxsize=1)
def _build_sc():
    mesh = plsc.VectorSubcoreMesh(core_axis_name="c", subcore_axis_name="s",
                                  num_cores=2, num_subcores=16)
    sc_params = pltpu.CompilerParams(needs_layout_passes=False)
    partition = pl.kernel(
        _partition_body,
        out_type=(
            jax.ShapeDtypeStruct((NTILES * 16,), jnp.int32),
            jax.ShapeDtypeStruct((NTILES * E_PAD,), jnp.int32),
            jax.ShapeDtypeStruct((NTILES * E_PAD,), jnp.int32),
            jax.ShapeDtypeStruct((NTILES * E_PAD,), jnp.float32),
        ),
        mesh=mesh,
        scratch_types=[
            pltpu.VMEM((CH,), jnp.int32),
            pltpu.VMEM((CH,), jnp.int32),
            pltpu.VMEM((CH,), jnp.float32),
            pltpu.VMEM((BUF,), jnp.int32),
            pltpu.VMEM((BUF,), jnp.int32),
            pltpu.VMEM((BUF,), jnp.float32),
            pltpu.VMEM((16,), jnp.int32),
        ],
        compiler_params=sc_params,
    )
    segmax = pl.kernel(
        _segmax_body,
        out_type=jax.ShapeDtypeStruct((N_PAD, DP), jnp.int32),
        mesh=mesh,
        scratch_types=[
            pltpu.VMEM((NPT, D), jnp.float32),
            pltpu.VMEM((NBUF, K, DP), jnp.int32),
            pltpu.VMEM((SPAN,), jnp.int32),
            pltpu.VMEM((SPAN,), jnp.int32),
            pltpu.VMEM((SPAN,), jnp.float32),
            pltpu.VMEM((16,), jnp.int32),
            pltpu.SemaphoreType.DMA((NBUF,)),
        ],
        compiler_params=sc_params,
    )
    return partition, segmax


def _dotT(a, w):
    return lax.dot_general(a, w, (((1,), (1,)), ((), ())),
                           preferred_element_type=jnp.float32)


def _pack_rows(p):
    u = lax.bitcast_convert_type(p, jnp.int32)
    r = (u + jnp.int32(0x7FFF)
         + (lax.shift_right_logical(u, 16) & 1)) >> 16
    return r[:, :DP] | (r[:, DP:] << 16)


def _unpack_rows(u):
    lo = lax.bitcast_convert_type(u << 16, jnp.float32)
    hi = lax.bitcast_convert_type(u & jnp.int32(-65536), jnp.float32)
    return jnp.concatenate([lo, hi], axis=1)


def _tc1_body(x_ref, w1_ref, b1_ref, wp_ref, bp_ref, ws_ref, bl_ref,
              h1_ref, p0_ref, s0_ref):
    h1 = jnp.tanh(_dotT(x_ref[...], w1_ref[...]) + b1_ref[0:1, :])
    p0 = jnp.maximum(_dotT(h1, wp_ref[...]) + bp_ref[0:1, :], 0.0)
    s0 = _dotT(h1, ws_ref[...]) + bl_ref[0:1, :]
    h1_ref[...] = h1
    p0_ref[...] = _pack_rows(p0)
    s0_ref[...] = s0


def _tc2_body(h1_ref, s0_ref, pooled_ref, wn_ref, wp_ref, bp_ref,
              ws_ref, bl_ref, h2_ref, p1_ref, s1_ref):
    pooled = _unpack_rows(pooled_ref[...])
    h2 = h1_ref[...] + jnp.tanh(s0_ref[...] + _dotT(pooled, wn_ref[...]))
    p1 = jnp.maximum(_dotT(h2, wp_ref[...]) + bp_ref[0:1, :], 0.0)
    s1 = _dotT(h2, ws_ref[...]) + bl_ref[0:1, :]
    h2_ref[...] = h2
    p1_ref[...] = _pack_rows(p1)
    s1_ref[...] = s1


def _tc3_body(h2_ref, s1_ref, pooled_ref, wn_ref, w2_ref, b2_ref,
              out_ref, mask_ref):
    h3 = h2_ref[...] + s1_ref[...] + _dotT(_unpack_rows(pooled_ref[...]), wn_ref[...])
    out8 = _dotT(jnp.tanh(h3), w2_ref[...]) + b2_ref[0:1, :]
    allz = jnp.all(h3 == 0.0, axis=1, keepdims=True)
    out_ref[...] = out8
    mask_ref[...] = jnp.broadcast_to(allz, (BM, 8)).astype(jnp.int32)


def _row_spec():
    return pl.BlockSpec((BM, D), lambda m: (m, 0))


def _full_spec(shape):
    return pl.BlockSpec(shape, lambda m: tuple(0 for _ in shape))


_tc1 = pl.pallas_call(
    _tc1_body,
    grid=(N // BM,),
    in_specs=[_row_spec(), _full_spec((D, D)), _full_spec((8, D)),
              _full_spec((D, D)), _full_spec((8, D)),
              _full_spec((D, D)), _full_spec((8, D))],
    out_specs=[_row_spec(), pl.BlockSpec((BM, DP), lambda m: (m, 0)),
               _row_spec()],
    out_shape=[jax.ShapeDtypeStruct((N, D), jnp.float32),
               jax.ShapeDtypeStruct((N, DP), jnp.int32),
               jax.ShapeDtypeStruct((N, D), jnp.float32)],
)

_tc2 = pl.pallas_call(
    _tc2_body,
    grid=(N // BM,),
    in_specs=[_row_spec(), _row_spec(), pl.BlockSpec((BM, DP), lambda m: (m, 0)),
              _full_spec((D, D)), _full_spec((D, D)), _full_spec((8, D)),
              _full_spec((D, D)), _full_spec((8, D))],
    out_specs=[_row_spec(), pl.BlockSpec((BM, DP), lambda m: (m, 0)),
               _row_spec()],
    out_shape=[jax.ShapeDtypeStruct((N, D), jnp.float32),
               jax.ShapeDtypeStruct((N, DP), jnp.int32),
               jax.ShapeDtypeStruct((N, D), jnp.float32)],
)

_tc3 = pl.pallas_call(
    _tc3_body,
    grid=(N // BM,),
    in_specs=[_row_spec(), _row_spec(), pl.BlockSpec((BM, DP), lambda m: (m, 0)),
              _full_spec((D, D)), _full_spec((8, D)), _full_spec((8, 8))],
    out_specs=[pl.BlockSpec((BM, 8), lambda m: (m, 0)),
               pl.BlockSpec((BM, 8), lambda m: (m, 0))],
    out_shape=[jax.ShapeDtypeStruct((N, 8), jnp.float32),
               jax.ShapeDtypeStruct((N, 8), jnp.int32)],
)


def _pad_rows(v, rows=8):
    return jnp.broadcast_to(v.reshape(1, -1), (rows, v.shape[0]))


def kernel(x, edge_index, edge_weight, W1, b1, Wp0, bp0, Ws0, Wn0, bl0,
           Wp1, bp1, Ws1, Wn1, bl1, W2, b2):
    _partition, _segmax = _build_sc()
    counts, esrc, edstl, eww = _partition(edge_index[0], edge_index[1],
                                          edge_weight)

    b1p, bp0p, bl0p = _pad_rows(b1), _pad_rows(bp0), _pad_rows(bl0)
    bp1p, bl1p = _pad_rows(bp1), _pad_rows(bl1)
    w2p = jnp.broadcast_to(W2, (8, D))
    b2p = jnp.broadcast_to(b2.reshape(1, 1), (8, 8))

    h1, p0, s0 = _tc1(x, W1, b1p, Wp0, bp0p, Ws0, bl0p)
    pooled0 = _segmax(p0, counts, esrc, edstl, eww)
    h2, p1, s1 = _tc2(h1, s0, pooled0, Wn0, Wp1, bp1p, Ws1, bl1p)
    pooled1 = _segmax(p1, counts, esrc, edstl, eww)
    out8, mask8 = _tc3(h2, s1, pooled1, Wn1, w2p, b2p)

    return out8[:, 0:1], mask8[:, 0].astype(bool)


# speedup vs baseline: 2.7060x; 1.2948x over previous
import functools

import jax
import jax.numpy as jnp
from jax import lax
from jax.experimental import pallas as pl
from jax.experimental.pallas import tpu as pltpu
from jax.experimental.pallas import tpu_sc as plsc

N = 10000
E = 160000
D = 256
NTILES = 32
NPT = 320
N_PAD = NTILES * NPT
CH = 3200
GRP = 8
FLUSH = 4096
BUF = FLUSH + GRP * 16
E_PAD = 40 * FLUSH
K = 128
DP = 128
NBUF = 2
SPAN = 2048
BM = 1000

def _partition_body(src_in, dst_in, ew, counts, esrc, edstl, eww,
                    srcc, dstc, wc, bsrc, bdst, bw, cbuf):
    wid = lax.axis_index("s") * 2 + lax.axis_index("c")
    base = wid * NPT
    ebase = wid * E_PAD

    zi = jnp.zeros((16,), jnp.int32)
    zf = jnp.zeros((16,), jnp.float32)

    def zero_b(i, _):
        bsrc[pl.ds(i * 16, 16)] = zi
        bdst[pl.ds(i * 16, 16)] = zi
        bw[pl.ds(i * 16, 16)] = zf
        return 0

    lax.fori_loop(0, BUF // 16, zero_b, 0)

    def chunk(c, carry):
        pltpu.sync_copy(src_in.at[pl.ds(c * CH, CH)], srcc)
        pltpu.sync_copy(dst_in.at[pl.ds(c * CH, CH)], dstc)
        pltpu.sync_copy(ew.at[pl.ds(c * CH, CH)], wc)

        def vstep(j, cy):
            cnt, hb = cy
            ms, offs, vss, vws, pcs = [], [], [], [], []
            for k in range(GRP):
                o = j * (GRP * 16) + k * 16
                vd = dstc[pl.ds(o, 16)]
                off = vd - base
                m = (off >= 0) & (off < NPT)
                ms.append(m)
                offs.append(off)
                vss.append(srcc[pl.ds(o, 16)])
                vws.append(wc[pl.ds(o, 16)])
                pcs.append(plsc.all_reduce_population_count(m)[0])
            for k in range(GRP):
                plsc.store_compressed(bsrc.at[pl.ds(cnt, 16)], vss[k],
                                      mask=ms[k])
                plsc.store_compressed(bdst.at[pl.ds(cnt, 16)], offs[k],
                                      mask=ms[k])
                plsc.store_compressed(bw.at[pl.ds(cnt, 16)], vws[k],
                                      mask=ms[k])
                cnt = cnt + pcs[k]

            def flush(cy3):
                cnt2, hb2 = cy3
                pltpu.sync_copy(bsrc.at[pl.ds(0, FLUSH)],
                                esrc.at[pl.ds(pl.multiple_of(ebase + hb2, FLUSH), FLUSH)])
                pltpu.sync_copy(bdst.at[pl.ds(0, FLUSH)],
                                edstl.at[pl.ds(pl.multiple_of(ebase + hb2, FLUSH), FLUSH)])
                pltpu.sync_copy(bw.at[pl.ds(0, FLUSH)],
                                eww.at[pl.ds(pl.multiple_of(ebase + hb2, FLUSH), FLUSH)])
                for t in range(GRP):
                    tsl = pl.ds(t * 16, 16)
                    fsl = pl.ds(FLUSH + t * 16, 16)
                    bsrc[tsl] = bsrc[fsl]
                    bdst[tsl] = bdst[fsl]
                    bw[tsl] = bw[fsl]
                return (cnt2 - FLUSH, hb2 + FLUSH)

            return lax.cond(cnt >= FLUSH, flush, lambda z: z, (cnt, hb))

        return lax.fori_loop(0, CH // (GRP * 16), vstep, carry)

    cnt, hb = lax.fori_loop(0, E // CH, chunk,
                            (jnp.int32(0), jnp.int32(0)))

    pltpu.sync_copy(bsrc.at[pl.ds(0, FLUSH)], esrc.at[pl.ds(pl.multiple_of(ebase + hb, FLUSH), FLUSH)])
    pltpu.sync_copy(bdst.at[pl.ds(0, FLUSH)], edstl.at[pl.ds(pl.multiple_of(ebase + hb, FLUSH), FLUSH)])
    pltpu.sync_copy(bw.at[pl.ds(0, FLUSH)], eww.at[pl.ds(pl.multiple_of(ebase + hb, FLUSH), FLUSH)])
    cbuf[...] = jnp.full((16,), hb + cnt, jnp.int32)
    pltpu.sync_copy(cbuf, counts.at[pl.ds(pl.multiple_of(wid * 16, 16), 16)])


def _segmax_body(p, counts, esrc, edstl, eww, pooled,
                 acc, bufs, sidx, sdst, sw, cbuf, sems):
    wid = lax.axis_index("s") * 2 + lax.axis_index("c")
    base = wid * NPT
    ebase = wid * E_PAD
    pltpu.sync_copy(counts.at[pl.ds(pl.multiple_of(wid * 16, 16), 16)], cbuf)
    count = cbuf[...][0]

    zf = jnp.zeros((16,), jnp.float32)

    def zr(r, _):
        for ci in range(D // 16):
            acc[r, pl.ds(ci * 16, 16)] = zf
        return 0

    lax.fori_loop(0, NPT, zr, 0)

    def accum(rows, ebeg):
        def grp(g, _):
            dv = sdst[pl.ds(ebeg + g * 16, 16)]
            w16 = sw[pl.ds(ebeg + g * 16, 16)]
            for lane in range(16):
                d = dv[lane]
                w = w16[lane]
                for ci in range(DP // 16):
                    v = rows[g * 16 + lane, pl.ds(ci * 16, 16)]
                    ra = plsc.bitcast(v << 16, jnp.float32)
                    rb = plsc.bitcast(v & jnp.int32(-65536), jnp.float32)
                    sa = pl.ds(ci * 16, 16)
                    sb = pl.ds(DP + ci * 16, 16)
                    acc[d, sa] = jnp.maximum(acc[d, sa], ra * w)
                    acc[d, sb] = jnp.maximum(acc[d, sb], rb * w)
            return 0

        lax.fori_loop(0, K // 16, grp, 0)

    def issue(t, b):
        off = pl.multiple_of(jnp.minimum(t, SPAN // K - 1) * K, K)
        pltpu.async_copy(p.at[sidx.at[pl.ds(off, K)]], bufs.at[b], sems.at[b])

    def drain(b):
        pltpu.make_async_copy(p.at[pl.ds(0, K)], bufs.at[b], sems.at[b]).wait()

    nspan = (count + (SPAN - 1)) // SPAN

    def span(s, _):
        soff = pl.multiple_of(ebase + s * SPAN, SPAN)
        pltpu.sync_copy(esrc.at[pl.ds(soff, SPAN)], sidx)
        pltpu.sync_copy(edstl.at[pl.ds(soff, SPAN)], sdst)
        pltpu.sync_copy(eww.at[pl.ds(soff, SPAN)], sw)
        rem = jnp.minimum(count - s * SPAN, SPAN)
        ng = (rem + (NBUF * K - 1)) // (NBUF * K)
        for b in range(NBUF):
            issue(jnp.int32(b), b)

        def ring(g, _):
            for b in range(NBUF):
                t = g * NBUF + b
                drain(b)
                accum(bufs.at[b], t * K)
                issue(t + NBUF, b)
            return 0

        lax.fori_loop(0, ng, ring, 0)
        for b in range(NBUF):
            drain(b)
        return 0

    lax.fori_loop(0, nspan, span, 0)

    WR = 64

    def rnd16(x):
        u = plsc.bitcast(x, jnp.int32)
        return (u + jnp.int32(0x7FFF)
                + (lax.shift_right_logical(u, 16) & 1)) >> 16

    for wchunk in range(NPT // WR):

        def wrow(r2, _):
            for ci in range(DP // 16):
                lo = rnd16(acc[wchunk * WR + r2, pl.ds(ci * 16, 16)])
                hi = rnd16(acc[wchunk * WR + r2, pl.ds(DP + ci * 16, 16)])
                bufs[0, r2, pl.ds(ci * 16, 16)] = lo | (hi << 16)
            return 0

        lax.fori_loop(0, WR, wrow, 0)
        pltpu.sync_copy(bufs.at[0].at[pl.ds(0, WR)],
                        pooled.at[pl.ds(base + wchunk * WR, WR)])


@functools.lru_cache(maxsize=1)
def _build_sc():
    mesh = plsc.VectorSubcoreMesh(core_axis_name="c", subcore_axis_name="s",
                                  num_cores=2, num_subcores=16)
    sc_params = pltpu.CompilerParams(needs_layout_passes=False)
    partition = pl.kernel(
        _partition_body,
        out_type=(
            jax.ShapeDtypeStruct((NTILES * 16,), jnp.int32),
            jax.ShapeDtypeStruct((NTILES * E_PAD,), jnp.int32),
            jax.ShapeDtypeStruct((NTILES * E_PAD,), jnp.int32),
            jax.ShapeDtypeStruct((NTILES * E_PAD,), jnp.float32),
        ),
        mesh=mesh,
        scratch_types=[
            pltpu.VMEM((CH,), jnp.int32),
            pltpu.VMEM((CH,), jnp.int32),
            pltpu.VMEM((CH,), jnp.float32),
            pltpu.VMEM((BUF,), jnp.int32),
            pltpu.VMEM((BUF,), jnp.int32),
            pltpu.VMEM((BUF,), jnp.float32),
            pltpu.VMEM((16,), jnp.int32),
        ],
        compiler_params=sc_params,
    )
    segmax = pl.kernel(
        _segmax_body,
        out_type=jax.ShapeDtypeStruct((N_PAD, DP), jnp.int32),
        mesh=mesh,
        scratch_types=[
            pltpu.VMEM((NPT, D), jnp.float32),
            pltpu.VMEM((NBUF, K, DP), jnp.int32),
            pltpu.VMEM((SPAN,), jnp.int32),
            pltpu.VMEM((SPAN,), jnp.int32),
            pltpu.VMEM((SPAN,), jnp.float32),
            pltpu.VMEM((16,), jnp.int32),
            pltpu.SemaphoreType.DMA((NBUF,)),
        ],
        compiler_params=sc_params,
    )
    return partition, segmax


def _dotT(a, w):
    return lax.dot_general(a, w, (((1,), (1,)), ((), ())),
                           preferred_element_type=jnp.float32)


def _pack_rows(p):
    u = lax.bitcast_convert_type(p, jnp.int32)
    r = (u + jnp.int32(0x7FFF)
         + (lax.shift_right_logical(u, 16) & 1)) >> 16
    return r[:, :DP] | (r[:, DP:] << 16)


def _unpack_rows(u):
    lo = lax.bitcast_convert_type(u << 16, jnp.float32)
    hi = lax.bitcast_convert_type(u & jnp.int32(-65536), jnp.float32)
    return jnp.concatenate([lo, hi], axis=1)


def _tc1_body(x_ref, w1_ref, b1_ref, wp_ref, bp_ref, ws_ref, bl_ref,
              h1_ref, p0_ref, s0_ref):
    h1 = jnp.tanh(_dotT(x_ref[...], w1_ref[...]) + b1_ref[0:1, :])
    p0 = jnp.maximum(_dotT(h1, wp_ref[...]) + bp_ref[0:1, :], 0.0)
    s0 = _dotT(h1, ws_ref[...]) + bl_ref[0:1, :]
    h1_ref[...] = h1
    p0_ref[...] = _pack_rows(p0)
    s0_ref[...] = s0


def _tc2_body(h1_ref, s0_ref, pooled_ref, wn_ref, wp_ref, bp_ref,
              ws_ref, bl_ref, h2_ref, p1_ref, s1_ref):
    pooled = _unpack_rows(pooled_ref[...])
    h2 = h1_ref[...] + jnp.tanh(s0_ref[...] + _dotT(pooled, wn_ref[...]))
    p1 = jnp.maximum(_dotT(h2, wp_ref[...]) + bp_ref[0:1, :], 0.0)
    s1 = _dotT(h2, ws_ref[...]) + bl_ref[0:1, :]
    h2_ref[...] = h2
    p1_ref[...] = _pack_rows(p1)
    s1_ref[...] = s1


def _tc3_body(h2_ref, s1_ref, pooled_ref, wn_ref, w2_ref, b2_ref,
              out_ref, mask_ref):
    h3 = h2_ref[...] + s1_ref[...] + _dotT(_unpack_rows(pooled_ref[...]), wn_ref[...])
    out8 = _dotT(jnp.tanh(h3), w2_ref[...]) + b2_ref[0:1, :]
    allz = jnp.all(h3 == 0.0, axis=1, keepdims=True)
    out_ref[...] = out8
    mask_ref[...] = jnp.broadcast_to(allz, (BM, 8)).astype(jnp.int32)


def _row_spec():
    return pl.BlockSpec((BM, D), lambda m: (m, 0))


def _full_spec(shape):
    return pl.BlockSpec(shape, lambda m: tuple(0 for _ in shape))


_tc1 = pl.pallas_call(
    _tc1_body,
    grid=(N // BM,),
    in_specs=[_row_spec(), _full_spec((D, D)), _full_spec((8, D)),
              _full_spec((D, D)), _full_spec((8, D)),
              _full_spec((D, D)), _full_spec((8, D))],
    out_specs=[_row_spec(), pl.BlockSpec((BM, DP), lambda m: (m, 0)),
               _row_spec()],
    out_shape=[jax.ShapeDtypeStruct((N, D), jnp.float32),
               jax.ShapeDtypeStruct((N, DP), jnp.int32),
               jax.ShapeDtypeStruct((N, D), jnp.float32)],
)

_tc2 = pl.pallas_call(
    _tc2_body,
    grid=(N // BM,),
    in_specs=[_row_spec(), _row_spec(), pl.BlockSpec((BM, DP), lambda m: (m, 0)),
              _full_spec((D, D)), _full_spec((D, D)), _full_spec((8, D)),
              _full_spec((D, D)), _full_spec((8, D))],
    out_specs=[_row_spec(), pl.BlockSpec((BM, DP), lambda m: (m, 0)),
               _row_spec()],
    out_shape=[jax.ShapeDtypeStruct((N, D), jnp.float32),
               jax.ShapeDtypeStruct((N, DP), jnp.int32),
               jax.ShapeDtypeStruct((N, D), jnp.float32)],
)

_tc3 = pl.pallas_call(
    _tc3_body,
    grid=(N // BM,),
    in_specs=[_row_spec(), _row_spec(), pl.BlockSpec((BM, DP), lambda m: (m, 0)),
              _full_spec((D, D)), _full_spec((8, D)), _full_spec((8, 8))],
    out_specs=[pl.BlockSpec((BM, 8), lambda m: (m, 0)),
               pl.BlockSpec((BM, 8), lambda m: (m, 0))],
    out_shape=[jax.ShapeDtypeStruct((N, 8), jnp.float32),
               jax.ShapeDtypeStruct((N, 8), jnp.int32)],
)


def _pad_rows(v, rows=8):
    return jnp.broadcast_to(v.reshape(1, -1), (rows, v.shape[0]))


def kernel(x, edge_index, edge_weight, W1, b1, Wp0, bp0, Ws0, Wn0, bl0,
           Wp1, bp1, Ws1, Wn1, bl1, W2, b2):
    _partition, _segmax = _build_sc()
    counts, esrc, edstl, eww = _partition(edge_index[0], edge_index[1],
                                          edge_weight)

    b1p, bp0p, bl0p = _pad_rows(b1), _pad_rows(bp0), _pad_rows(bl0)
    bp1p, bl1p = _pad_rows(bp1), _pad_rows(bl1)
    w2p = jnp.broadcast_to(W2, (8, D))
    b2p = jnp.broadcast_to(b2.reshape(1, 1), (8, 8))

    h1, p0, s0 = _tc1(x, W1, b1p, Wp0, bp0p, Ws0, bl0p)
    pooled0 = _segmax(p0, counts, esrc, edstl, eww)
    h2, p1, s1 = _tc2(h1, s0, pooled0, Wn0, Wp1, bp1p, Ws1, bl1p)
    pooled1 = _segmax(p1, counts, esrc, edstl, eww)
    out8, mask8 = _tc3(h2, s1, pooled1, Wn1, w2p, b2p)

    return out8[:, 0:1], mask8[:, 0].astype(bool)


# speedup vs baseline: 2.7190x; 1.0048x over previous
import functools

import jax
import jax.numpy as jnp
from jax import lax
from jax.experimental import pallas as pl
from jax.experimental.pallas import tpu as pltpu
from jax.experimental.pallas import tpu_sc as plsc

N = 10000
E = 160000
D = 256
NTILES = 32
NPT = 320
N_PAD = NTILES * NPT
CH = 3200
GRP = 8
FLUSH = 4096
BUF = FLUSH + GRP * 16
E_PAD = 40 * FLUSH
K = 128
DP = 128
NBUF = 2
SPAN = 2048
BM = 1000

def _partition_body(src_in, dst_in, ew, counts, esrc, edstl, eww,
                    srcc, dstc, wc, bsrc, bdst, bw, cbuf):
    wid = lax.axis_index("s") * 2 + lax.axis_index("c")
    base = wid * NPT
    ebase = wid * E_PAD

    zi = jnp.zeros((16,), jnp.int32)
    zf = jnp.zeros((16,), jnp.float32)

    def zero_b(i, _):
        bsrc[pl.ds(i * 16, 16)] = zi
        bdst[pl.ds(i * 16, 16)] = zi
        bw[pl.ds(i * 16, 16)] = zf
        return 0

    lax.fori_loop(0, BUF // 16, zero_b, 0)

    def chunk(c, carry):
        pltpu.sync_copy(src_in.at[pl.ds(c * CH, CH)], srcc)
        pltpu.sync_copy(dst_in.at[pl.ds(c * CH, CH)], dstc)
        pltpu.sync_copy(ew.at[pl.ds(c * CH, CH)], wc)

        def vstep(j, cy):
            cnt, hb = cy
            ms, offs, vss, vws, pcs = [], [], [], [], []
            for k in range(GRP):
                o = j * (GRP * 16) + k * 16
                vd = dstc[pl.ds(o, 16)]
                off = vd - base
                m = (off >= 0) & (off < NPT)
                ms.append(m)
                offs.append(off)
                vss.append(srcc[pl.ds(o, 16)])
                vws.append(wc[pl.ds(o, 16)])
                pcs.append(plsc.all_reduce_population_count(m)[0])
            for k in range(GRP):
                plsc.store_compressed(bsrc.at[pl.ds(cnt, 16)], vss[k],
                                      mask=ms[k])
                plsc.store_compressed(bdst.at[pl.ds(cnt, 16)], offs[k],
                                      mask=ms[k])
                plsc.store_compressed(bw.at[pl.ds(cnt, 16)], vws[k],
                                      mask=ms[k])
                cnt = cnt + pcs[k]

            def flush(cy3):
                cnt2, hb2 = cy3
                pltpu.sync_copy(bsrc.at[pl.ds(0, FLUSH)],
                                esrc.at[pl.ds(pl.multiple_of(ebase + hb2, FLUSH), FLUSH)])
                pltpu.sync_copy(bdst.at[pl.ds(0, FLUSH)],
                                edstl.at[pl.ds(pl.multiple_of(ebase + hb2, FLUSH), FLUSH)])
                pltpu.sync_copy(bw.at[pl.ds(0, FLUSH)],
                                eww.at[pl.ds(pl.multiple_of(ebase + hb2, FLUSH), FLUSH)])
                for t in range(GRP):
                    tsl = pl.ds(t * 16, 16)
                    fsl = pl.ds(FLUSH + t * 16, 16)
                    bsrc[tsl] = bsrc[fsl]
                    bdst[tsl] = bdst[fsl]
                    bw[tsl] = bw[fsl]
                return (cnt2 - FLUSH, hb2 + FLUSH)

            return lax.cond(cnt >= FLUSH, flush, lambda z: z, (cnt, hb))

        return lax.fori_loop(0, CH // (GRP * 16), vstep, carry)

    cnt, hb = lax.fori_loop(0, E // CH, chunk,
                            (jnp.int32(0), jnp.int32(0)))

    pltpu.sync_copy(bsrc.at[pl.ds(0, FLUSH)], esrc.at[pl.ds(pl.multiple_of(ebase + hb, FLUSH), FLUSH)])
    pltpu.sync_copy(bdst.at[pl.ds(0, FLUSH)], edstl.at[pl.ds(pl.multiple_of(ebase + hb, FLUSH), FLUSH)])
    pltpu.sync_copy(bw.at[pl.ds(0, FLUSH)], eww.at[pl.ds(pl.multiple_of(ebase + hb, FLUSH), FLUSH)])
    cbuf[...] = jnp.full((16,), hb + cnt, jnp.int32)
    pltpu.sync_copy(cbuf, counts.at[pl.ds(pl.multiple_of(wid * 16, 16), 16)])


def _segmax_body(p, counts, esrc, edstl, eww, pooled,
                 acc, bufs, sidx, sdst, sw, cbuf, sems):
    wid = lax.axis_index("s") * 2 + lax.axis_index("c")
    base = wid * NPT
    ebase = wid * E_PAD
    pltpu.sync_copy(counts.at[pl.ds(pl.multiple_of(wid * 16, 16), 16)], cbuf)
    count = cbuf[...][0]

    zf = jnp.zeros((16,), jnp.float32)

    def zr(r, _):
        for ci in range(D // 16):
            acc[r, pl.ds(ci * 16, 16)] = zf
        return 0

    lax.fori_loop(0, NPT, zr, 0)

    def accum(rows, ebeg):
        def grp(g, _):
            dv = sdst[pl.ds(ebeg + g * 16, 16)]
            w16 = sw[pl.ds(ebeg + g * 16, 16)]
            ds_ = [dv[lane] for lane in range(16)]
            for lane in range(16):
                d = ds_[lane]
                w = w16[lane]
                for ci in range(DP // 16):
                    v = rows[g * 16 + lane, pl.ds(ci * 16, 16)]
                    ra = plsc.bitcast(v << 16, jnp.float32)
                    rb = plsc.bitcast(v & jnp.int32(-65536), jnp.float32)
                    sa = pl.ds(ci * 16, 16)
                    sb = pl.ds(DP + ci * 16, 16)
                    acc[d, sa] = jnp.maximum(acc[d, sa], ra * w)
                    acc[d, sb] = jnp.maximum(acc[d, sb], rb * w)
            return 0

        lax.fori_loop(0, K // 16, grp, 0)

    def issue(t, b):
        off = pl.multiple_of(jnp.minimum(t, SPAN // K - 1) * K, K)
        pltpu.async_copy(p.at[sidx.at[pl.ds(off, K)]], bufs.at[b], sems.at[b])

    def drain(b):
        pltpu.make_async_copy(p.at[pl.ds(0, K)], bufs.at[b], sems.at[b]).wait()

    nspan = (count + (SPAN - 1)) // SPAN

    def span(s, _):
        soff = pl.multiple_of(ebase + s * SPAN, SPAN)
        pltpu.sync_copy(esrc.at[pl.ds(soff, SPAN)], sidx)
        pltpu.sync_copy(edstl.at[pl.ds(soff, SPAN)], sdst)
        pltpu.sync_copy(eww.at[pl.ds(soff, SPAN)], sw)
        rem = jnp.minimum(count - s * SPAN, SPAN)
        ng = (rem + (NBUF * K - 1)) // (NBUF * K)
        for b in range(NBUF):
            issue(jnp.int32(b), b)

        def ring(g, _):
            for b in range(NBUF):
                t = g * NBUF + b
                drain(b)
                accum(bufs.at[b], t * K)
                issue(t + NBUF, b)
            return 0

        lax.fori_loop(0, ng, ring, 0)
        for b in range(NBUF):
            drain(b)
        return 0

    lax.fori_loop(0, nspan, span, 0)

    WR = 64

    def rnd16(x):
        u = plsc.bitcast(x, jnp.int32)
        return (u + jnp.int32(0x7FFF)
                + (lax.shift_right_logical(u, 16) & 1)) >> 16

    for wchunk in range(NPT // WR):

        def wrow(r2, _):
            for ci in range(DP // 16):
                lo = rnd16(acc[wchunk * WR + r2, pl.ds(ci * 16, 16)])
                hi = rnd16(acc[wchunk * WR + r2, pl.ds(DP + ci * 16, 16)])
                bufs[0, r2, pl.ds(ci * 16, 16)] = lo | (hi << 16)
            return 0

        lax.fori_loop(0, WR, wrow, 0)
        pltpu.sync_copy(bufs.at[0].at[pl.ds(0, WR)],
                        pooled.at[pl.ds(base + wchunk * WR, WR)])


@functools.lru_cache(maxsize=1)
def _build_sc():
    mesh = plsc.VectorSubcoreMesh(core_axis_name="c", subcore_axis_name="s",
                                  num_cores=2, num_subcores=16)
    sc_params = pltpu.CompilerParams(needs_layout_passes=False)
    partition = pl.kernel(
        _partition_body,
        out_type=(
            jax.ShapeDtypeStruct((NTILES * 16,), jnp.int32),
            jax.ShapeDtypeStruct((NTILES * E_PAD,), jnp.int32),
            jax.ShapeDtypeStruct((NTILES * E_PAD,), jnp.int32),
            jax.ShapeDtypeStruct((NTILES * E_PAD,), jnp.float32),
        ),
        mesh=mesh,
        scratch_types=[
            pltpu.VMEM((CH,), jnp.int32),
            pltpu.VMEM((CH,), jnp.int32),
            pltpu.VMEM((CH,), jnp.float32),
            pltpu.VMEM((BUF,), jnp.int32),
            pltpu.VMEM((BUF,), jnp.int32),
            pltpu.VMEM((BUF,), jnp.float32),
            pltpu.VMEM((16,), jnp.int32),
        ],
        compiler_params=sc_params,
    )
    segmax = pl.kernel(
        _segmax_body,
        out_type=jax.ShapeDtypeStruct((N_PAD, DP), jnp.int32),
        mesh=mesh,
        scratch_types=[
            pltpu.VMEM((NPT, D), jnp.float32),
            pltpu.VMEM((NBUF, K, DP), jnp.int32),
            pltpu.VMEM((SPAN,), jnp.int32),
            pltpu.VMEM((SPAN,), jnp.int32),
            pltpu.VMEM((SPAN,), jnp.float32),
            pltpu.VMEM((16,), jnp.int32),
            pltpu.SemaphoreType.DMA((NBUF,)),
        ],
        compiler_params=sc_params,
    )
    return partition, segmax


def _dotT(a, w):
    return lax.dot_general(a, w, (((1,), (1,)), ((), ())),
                           preferred_element_type=jnp.float32)


def _pack_rows(p):
    u = lax.bitcast_convert_type(p, jnp.int32)
    r = (u + jnp.int32(0x7FFF)
         + (lax.shift_right_logical(u, 16) & 1)) >> 16
    return r[:, :DP] | (r[:, DP:] << 16)


def _unpack_rows(u):
    lo = lax.bitcast_convert_type(u << 16, jnp.float32)
    hi = lax.bitcast_convert_type(u & jnp.int32(-65536), jnp.float32)
    return jnp.concatenate([lo, hi], axis=1)


def _tc1_body(x_ref, w1_ref, b1_ref, wp_ref, bp_ref, ws_ref, bl_ref,
              h1_ref, p0_ref, s0_ref):
    h1 = jnp.tanh(_dotT(x_ref[...], w1_ref[...]) + b1_ref[0:1, :])
    p0 = jnp.maximum(_dotT(h1, wp_ref[...]) + bp_ref[0:1, :], 0.0)
    s0 = _dotT(h1, ws_ref[...]) + bl_ref[0:1, :]
    h1_ref[...] = h1
    p0_ref[...] = _pack_rows(p0)
    s0_ref[...] = s0


def _tc2_body(h1_ref, s0_ref, pooled_ref, wn_ref, wp_ref, bp_ref,
              ws_ref, bl_ref, h2_ref, p1_ref, s1_ref):
    pooled = _unpack_rows(pooled_ref[...])
    h2 = h1_ref[...] + jnp.tanh(s0_ref[...] + _dotT(pooled, wn_ref[...]))
    p1 = jnp.maximum(_dotT(h2, wp_ref[...]) + bp_ref[0:1, :], 0.0)
    s1 = _dotT(h2, ws_ref[...]) + bl_ref[0:1, :]
    h2_ref[...] = h2
    p1_ref[...] = _pack_rows(p1)
    s1_ref[...] = s1


def _tc3_body(h2_ref, s1_ref, pooled_ref, wn_ref, w2_ref, b2_ref,
              out_ref, mask_ref):
    h3 = h2_ref[...] + s1_ref[...] + _dotT(_unpack_rows(pooled_ref[...]), wn_ref[...])
    out8 = _dotT(jnp.tanh(h3), w2_ref[...]) + b2_ref[0:1, :]
    allz = jnp.all(h3 == 0.0, axis=1, keepdims=True)
    out_ref[...] = out8
    mask_ref[...] = jnp.broadcast_to(allz, (BM, 8)).astype(jnp.int32)


def _row_spec():
    return pl.BlockSpec((BM, D), lambda m: (m, 0))


def _full_spec(shape):
    return pl.BlockSpec(shape, lambda m: tuple(0 for _ in shape))


_tc1 = pl.pallas_call(
    _tc1_body,
    grid=(N // BM,),
    in_specs=[_row_spec(), _full_spec((D, D)), _full_spec((8, D)),
              _full_spec((D, D)), _full_spec((8, D)),
              _full_spec((D, D)), _full_spec((8, D))],
    out_specs=[_row_spec(), pl.BlockSpec((BM, DP), lambda m: (m, 0)),
               _row_spec()],
    out_shape=[jax.ShapeDtypeStruct((N, D), jnp.float32),
               jax.ShapeDtypeStruct((N, DP), jnp.int32),
               jax.ShapeDtypeStruct((N, D), jnp.float32)],
)

_tc2 = pl.pallas_call(
    _tc2_body,
    grid=(N // BM,),
    in_specs=[_row_spec(), _row_spec(), pl.BlockSpec((BM, DP), lambda m: (m, 0)),
              _full_spec((D, D)), _full_spec((D, D)), _full_spec((8, D)),
              _full_spec((D, D)), _full_spec((8, D))],
    out_specs=[_row_spec(), pl.BlockSpec((BM, DP), lambda m: (m, 0)),
               _row_spec()],
    out_shape=[jax.ShapeDtypeStruct((N, D), jnp.float32),
               jax.ShapeDtypeStruct((N, DP), jnp.int32),
               jax.ShapeDtypeStruct((N, D), jnp.float32)],
)

_tc3 = pl.pallas_call(
    _tc3_body,
    grid=(N // BM,),
    in_specs=[_row_spec(), _row_spec(), pl.BlockSpec((BM, DP), lambda m: (m, 0)),
              _full_spec((D, D)), _full_spec((8, D)), _full_spec((8, 8))],
    out_specs=[pl.BlockSpec((BM, 8), lambda m: (m, 0)),
               pl.BlockSpec((BM, 8), lambda m: (m, 0))],
    out_shape=[jax.ShapeDtypeStruct((N, 8), jnp.float32),
               jax.ShapeDtypeStruct((N, 8), jnp.int32)],
)


def _pad_rows(v, rows=8):
    return jnp.broadcast_to(v.reshape(1, -1), (rows, v.shape[0]))


def kernel(x, edge_index, edge_weight, W1, b1, Wp0, bp0, Ws0, Wn0, bl0,
           Wp1, bp1, Ws1, Wn1, bl1, W2, b2):
    _partition, _segmax = _build_sc()
    counts, esrc, edstl, eww = _partition(edge_index[0], edge_index[1],
                                          edge_weight)

    b1p, bp0p, bl0p = _pad_rows(b1), _pad_rows(bp0), _pad_rows(bl0)
    bp1p, bl1p = _pad_rows(bp1), _pad_rows(bl1)
    w2p = jnp.broadcast_to(W2, (8, D))
    b2p = jnp.broadcast_to(b2.reshape(1, 1), (8, 8))

    h1, p0, s0 = _tc1(x, W1, b1p, Wp0, bp0p, Ws0, bl0p)
    pooled0 = _segmax(p0, counts, esrc, edstl, eww)
    h2, p1, s1 = _tc2(h1, s0, pooled0, Wn0, Wp1, bp1p, Ws1, bl1p)
    pooled1 = _segmax(p1, counts, esrc, edstl, eww)
    out8, mask8 = _tc3(h2, s1, pooled1, Wn1, w2p, b2p)

    return out8[:, 0:1], mask8[:, 0].astype(bool)


# speedup vs baseline: 3.4550x; 1.2707x over previous
import functools

import jax
import jax.numpy as jnp
from jax import lax
from jax.experimental import pallas as pl
from jax.experimental.pallas import tpu as pltpu
from jax.experimental.pallas import tpu_sc as plsc

N = 10000
E = 160000
D = 256
NTILES = 32
NPT = 320
N_PAD = NTILES * NPT
CH = 3200
GRP = 8
FLUSH = 4096
BUF = FLUSH + GRP * 16
E_PAD = 40 * FLUSH
K = 128
DP = 128
NBUF = 2
SPAN = 2048
BM = 1000

def _partition_body(src_in, dst_in, ew, counts, esrc, edstl, eww,
                    srcc, dstc, wc, bsrc, bdst, bw, cbuf):
    wid = lax.axis_index("s") * 2 + lax.axis_index("c")
    base = wid * NPT
    ebase = wid * E_PAD

    zi = jnp.zeros((16,), jnp.int32)

    def zero_b(i, _):
        bsrc[pl.ds(i * 16, 16)] = zi
        bdst[pl.ds(i * 16, 16)] = zi
        bw[pl.ds(i * 16, 16)] = zi
        return 0

    lax.fori_loop(0, BUF // 16, zero_b, 0)

    def chunk(c, carry):
        pltpu.sync_copy(src_in.at[pl.ds(c * CH, CH)], srcc)
        pltpu.sync_copy(dst_in.at[pl.ds(c * CH, CH)], dstc)
        pltpu.sync_copy(ew.at[pl.ds(c * CH, CH)], wc)

        def vstep(j, cy):
            cnt, hb = cy
            ms, offs, vss, vws, pcs = [], [], [], [], []
            for k in range(GRP):
                o = j * (GRP * 16) + k * 16
                vd = dstc[pl.ds(o, 16)]
                off = vd - base
                m = (off >= 0) & (off < NPT)
                ms.append(m)
                offs.append(off)
                vss.append(srcc[pl.ds(o, 16)])
                uw = plsc.bitcast(wc[pl.ds(o, 16)], jnp.int32)
                wr = (uw + jnp.int32(0x7FFF)
                      + (lax.shift_right_logical(uw, 16) & 1)) >> 16
                vws.append(wr | (wr << 16))
                pcs.append(plsc.all_reduce_population_count(m)[0])
            for k in range(GRP):
                plsc.store_compressed(bsrc.at[pl.ds(cnt, 16)], vss[k],
                                      mask=ms[k])
                plsc.store_compressed(bdst.at[pl.ds(cnt, 16)], offs[k],
                                      mask=ms[k])
                plsc.store_compressed(bw.at[pl.ds(cnt, 16)], vws[k],
                                      mask=ms[k])
                cnt = cnt + pcs[k]

            def flush(cy3):
                cnt2, hb2 = cy3
                pltpu.sync_copy(bsrc.at[pl.ds(0, FLUSH)],
                                esrc.at[pl.ds(pl.multiple_of(ebase + hb2, FLUSH), FLUSH)])
                pltpu.sync_copy(bdst.at[pl.ds(0, FLUSH)],
                                edstl.at[pl.ds(pl.multiple_of(ebase + hb2, FLUSH), FLUSH)])
                pltpu.sync_copy(bw.at[pl.ds(0, FLUSH)],
                                eww.at[pl.ds(pl.multiple_of(ebase + hb2, FLUSH), FLUSH)])
                for t in range(GRP):
                    tsl = pl.ds(t * 16, 16)
                    fsl = pl.ds(FLUSH + t * 16, 16)
                    bsrc[tsl] = bsrc[fsl]
                    bdst[tsl] = bdst[fsl]
                    bw[tsl] = bw[fsl]
                return (cnt2 - FLUSH, hb2 + FLUSH)

            return lax.cond(cnt >= FLUSH, flush, lambda z: z, (cnt, hb))

        return lax.fori_loop(0, CH // (GRP * 16), vstep, carry)

    cnt, hb = lax.fori_loop(0, E // CH, chunk,
                            (jnp.int32(0), jnp.int32(0)))

    pltpu.sync_copy(bsrc.at[pl.ds(0, FLUSH)], esrc.at[pl.ds(pl.multiple_of(ebase + hb, FLUSH), FLUSH)])
    pltpu.sync_copy(bdst.at[pl.ds(0, FLUSH)], edstl.at[pl.ds(pl.multiple_of(ebase + hb, FLUSH), FLUSH)])
    pltpu.sync_copy(bw.at[pl.ds(0, FLUSH)], eww.at[pl.ds(pl.multiple_of(ebase + hb, FLUSH), FLUSH)])
    cbuf[...] = jnp.full((16,), hb + cnt, jnp.int32)
    pltpu.sync_copy(cbuf, counts.at[pl.ds(pl.multiple_of(wid * 16, 16), 16)])


def _segmax_body(p, counts, esrc, edstl, eww, pooled,
                 acc, bufs, sidx, sdst, sw, cbuf, sems):
    wid = lax.axis_index("s") * 2 + lax.axis_index("c")
    base = wid * NPT
    ebase = wid * E_PAD
    pltpu.sync_copy(counts.at[pl.ds(pl.multiple_of(wid * 16, 16), 16)], cbuf)
    count = cbuf[...][0]

    zi = jnp.zeros((16,), jnp.int32)

    def zr(r, _):
        for ci in range(DP // 16):
            acc[r, pl.ds(ci * 16, 16)] = zi
        return 0

    lax.fori_loop(0, NPT, zr, 0)

    def accum(rows, ebeg):
        def grp(g, _):
            dv = sdst[pl.ds(ebeg + g * 16, 16)]
            w16 = sw[pl.ds(ebeg + g * 16, 16)]
            ds_ = [dv[lane] for lane in range(16)]
            for lane in range(16):
                d = ds_[lane]
                wb = plsc.bitcast(jnp.broadcast_to(w16[lane], (16,)),
                                  jnp.bfloat16)
                for ci in range(DP // 16):
                    sl = pl.ds(ci * 16, 16)
                    rv = plsc.bitcast(rows[g * 16 + lane, sl], jnp.bfloat16)
                    av = plsc.bitcast(acc[d, sl], jnp.bfloat16)
                    nx = jnp.maximum(av, rv * wb)
                    acc[d, sl] = plsc.bitcast(nx, jnp.int32)
            return 0

        lax.fori_loop(0, K // 16, grp, 0)

    def issue(t, b):
        off = pl.multiple_of(jnp.minimum(t, SPAN // K - 1) * K, K)
        pltpu.async_copy(p.at[sidx.at[pl.ds(off, K)]], bufs.at[b], sems.at[b])

    def drain(b):
        pltpu.make_async_copy(p.at[pl.ds(0, K)], bufs.at[b], sems.at[b]).wait()

    nspan = (count + (SPAN - 1)) // SPAN

    def span(s, _):
        soff = pl.multiple_of(ebase + s * SPAN, SPAN)
        pltpu.sync_copy(esrc.at[pl.ds(soff, SPAN)], sidx)
        pltpu.sync_copy(edstl.at[pl.ds(soff, SPAN)], sdst)
        pltpu.sync_copy(eww.at[pl.ds(soff, SPAN)], sw)
        rem = jnp.minimum(count - s * SPAN, SPAN)
        ng = (rem + (NBUF * K - 1)) // (NBUF * K)
        for b in range(NBUF):
            issue(jnp.int32(b), b)

        def ring(g, _):
            for b in range(NBUF):
                t = g * NBUF + b
                drain(b)
                accum(bufs.at[b], t * K)
                issue(t + NBUF, b)
            return 0

        lax.fori_loop(0, ng, ring, 0)
        for b in range(NBUF):
            drain(b)
        return 0

    lax.fori_loop(0, nspan, span, 0)

    pltpu.sync_copy(acc, pooled.at[pl.ds(base, NPT)])


@functools.lru_cache(maxsize=1)
def _build_sc():
    mesh = plsc.VectorSubcoreMesh(core_axis_name="c", subcore_axis_name="s",
                                  num_cores=2, num_subcores=16)
    sc_params = pltpu.CompilerParams(needs_layout_passes=False)
    partition = pl.kernel(
        _partition_body,
        out_type=(
            jax.ShapeDtypeStruct((NTILES * 16,), jnp.int32),
            jax.ShapeDtypeStruct((NTILES * E_PAD,), jnp.int32),
            jax.ShapeDtypeStruct((NTILES * E_PAD,), jnp.int32),
            jax.ShapeDtypeStruct((NTILES * E_PAD,), jnp.int32),
        ),
        mesh=mesh,
        scratch_types=[
            pltpu.VMEM((CH,), jnp.int32),
            pltpu.VMEM((CH,), jnp.int32),
            pltpu.VMEM((CH,), jnp.float32),
            pltpu.VMEM((BUF,), jnp.int32),
            pltpu.VMEM((BUF,), jnp.int32),
            pltpu.VMEM((BUF,), jnp.int32),
            pltpu.VMEM((16,), jnp.int32),
        ],
        compiler_params=sc_params,
    )
    segmax = pl.kernel(
        _segmax_body,
        out_type=jax.ShapeDtypeStruct((N_PAD, DP), jnp.int32),
        mesh=mesh,
        scratch_types=[
            pltpu.VMEM((NPT, DP), jnp.int32),
            pltpu.VMEM((NBUF, K, DP), jnp.int32),
            pltpu.VMEM((SPAN,), jnp.int32),
            pltpu.VMEM((SPAN,), jnp.int32),
            pltpu.VMEM((SPAN,), jnp.int32),
            pltpu.VMEM((16,), jnp.int32),
            pltpu.SemaphoreType.DMA((NBUF,)),
        ],
        compiler_params=sc_params,
    )
    return partition, segmax


def _dotT(a, w):
    return lax.dot_general(a, w, (((1,), (1,)), ((), ())),
                           preferred_element_type=jnp.float32)


def _pack_rows(p):
    u = lax.bitcast_convert_type(p, jnp.int32)
    r = (u + jnp.int32(0x7FFF)
         + (lax.shift_right_logical(u, 16) & 1)) >> 16
    return r[:, :DP] | (r[:, DP:] << 16)


def _unpack_rows(u):
    lo = lax.bitcast_convert_type(u << 16, jnp.float32)
    hi = lax.bitcast_convert_type(u & jnp.int32(-65536), jnp.float32)
    return jnp.concatenate([lo, hi], axis=1)


def _tc1_body(x_ref, w1_ref, b1_ref, wp_ref, bp_ref, ws_ref, bl_ref,
              h1_ref, p0_ref, s0_ref):
    h1 = jnp.tanh(_dotT(x_ref[...], w1_ref[...]) + b1_ref[0:1, :])
    p0 = jnp.maximum(_dotT(h1, wp_ref[...]) + bp_ref[0:1, :], 0.0)
    s0 = _dotT(h1, ws_ref[...]) + bl_ref[0:1, :]
    h1_ref[...] = h1
    p0_ref[...] = _pack_rows(p0)
    s0_ref[...] = s0


def _tc2_body(h1_ref, s0_ref, pooled_ref, wn_ref, wp_ref, bp_ref,
              ws_ref, bl_ref, h2_ref, p1_ref, s1_ref):
    pooled = _unpack_rows(pooled_ref[...])
    h2 = h1_ref[...] + jnp.tanh(s0_ref[...] + _dotT(pooled, wn_ref[...]))
    p1 = jnp.maximum(_dotT(h2, wp_ref[...]) + bp_ref[0:1, :], 0.0)
    s1 = _dotT(h2, ws_ref[...]) + bl_ref[0:1, :]
    h2_ref[...] = h2
    p1_ref[...] = _pack_rows(p1)
    s1_ref[...] = s1


def _tc3_body(h2_ref, s1_ref, pooled_ref, wn_ref, w2_ref, b2_ref,
              out_ref, mask_ref):
    h3 = h2_ref[...] + s1_ref[...] + _dotT(_unpack_rows(pooled_ref[...]), wn_ref[...])
    out8 = _dotT(jnp.tanh(h3), w2_ref[...]) + b2_ref[0:1, :]
    allz = jnp.all(h3 == 0.0, axis=1, keepdims=True)
    out_ref[...] = out8
    mask_ref[...] = jnp.broadcast_to(allz, (BM, 8)).astype(jnp.int32)


def _row_spec():
    return pl.BlockSpec((BM, D), lambda m: (m, 0))


def _full_spec(shape):
    return pl.BlockSpec(shape, lambda m: tuple(0 for _ in shape))


_tc1 = pl.pallas_call(
    _tc1_body,
    grid=(N // BM,),
    in_specs=[_row_spec(), _full_spec((D, D)), _full_spec((8, D)),
              _full_spec((D, D)), _full_spec((8, D)),
              _full_spec((D, D)), _full_spec((8, D))],
    out_specs=[_row_spec(), pl.BlockSpec((BM, DP), lambda m: (m, 0)),
               _row_spec()],
    out_shape=[jax.ShapeDtypeStruct((N, D), jnp.float32),
               jax.ShapeDtypeStruct((N, DP), jnp.int32),
               jax.ShapeDtypeStruct((N, D), jnp.float32)],
)

_tc2 = pl.pallas_call(
    _tc2_body,
    grid=(N // BM,),
    in_specs=[_row_spec(), _row_spec(), pl.BlockSpec((BM, DP), lambda m: (m, 0)),
              _full_spec((D, D)), _full_spec((D, D)), _full_spec((8, D)),
              _full_spec((D, D)), _full_spec((8, D))],
    out_specs=[_row_spec(), pl.BlockSpec((BM, DP), lambda m: (m, 0)),
               _row_spec()],
    out_shape=[jax.ShapeDtypeStruct((N, D), jnp.float32),
               jax.ShapeDtypeStruct((N, DP), jnp.int32),
               jax.ShapeDtypeStruct((N, D), jnp.float32)],
)

_tc3 = pl.pallas_call(
    _tc3_body,
    grid=(N // BM,),
    in_specs=[_row_spec(), _row_spec(), pl.BlockSpec((BM, DP), lambda m: (m, 0)),
              _full_spec((D, D)), _full_spec((8, D)), _full_spec((8, 8))],
    out_specs=[pl.BlockSpec((BM, 8), lambda m: (m, 0)),
               pl.BlockSpec((BM, 8), lambda m: (m, 0))],
    out_shape=[jax.ShapeDtypeStruct((N, 8), jnp.float32),
               jax.ShapeDtypeStruct((N, 8), jnp.int32)],
)


def _pad_rows(v, rows=8):
    return jnp.broadcast_to(v.reshape(1, -1), (rows, v.shape[0]))


def kernel(x, edge_index, edge_weight, W1, b1, Wp0, bp0, Ws0, Wn0, bl0,
           Wp1, bp1, Ws1, Wn1, bl1, W2, b2):
    _partition, _segmax = _build_sc()
    counts, esrc, edstl, eww = _partition(edge_index[0], edge_index[1],
                                          edge_weight)

    b1p, bp0p, bl0p = _pad_rows(b1), _pad_rows(bp0), _pad_rows(bl0)
    bp1p, bl1p = _pad_rows(bp1), _pad_rows(bl1)
    w2p = jnp.broadcast_to(W2, (8, D))
    b2p = jnp.broadcast_to(b2.reshape(1, 1), (8, 8))

    h1, p0, s0 = _tc1(x, W1, b1p, Wp0, bp0p, Ws0, bl0p)
    pooled0 = _segmax(p0, counts, esrc, edstl, eww)
    h2, p1, s1 = _tc2(h1, s0, pooled0, Wn0, Wp1, bp1p, Ws1, bl1p)
    pooled1 = _segmax(p1, counts, esrc, edstl, eww)
    out8, mask8 = _tc3(h2, s1, pooled1, Wn1, w2p, b2p)

    return out8[:, 0:1], mask8[:, 0].astype(bool)
